# Initial kernel scaffold; baseline (speedup 1.0000x reference)
import jax
import jax.numpy as jnp
from jax.experimental import pallas as pl


def kernel(x, edge_index, batch, params):
    raise NotImplementedError("write your pallas kernel here")


# speedup vs baseline: 8.3983x; 8.3983x over previous
import functools

import jax
import jax.numpy as jnp
from jax import lax
from jax.experimental import pallas as pl
from jax.experimental.pallas import tpu as pltpu
from jax.experimental.pallas import tpu_sc as plsc

_N = 10000
_E = 320000
_H = 128
_G = 64
_CLS = 10
_NC = 2
_NS = 16
_L = 16
_NW = _NC * _NS
_EPT = _E // _NW
_EB = 80
_NB = _EPT // _EB
_NPAIR = (_NB - 1) // 2
_DL = _H // _L
_DR = 80
_NPAD = _DR * _H
_DRT = _DR // _NS
_APT = _NPAD // _NS
_ISQ = 1.0 / float(_H) ** 0.5


def _sc_mesh():
    return plsc.VectorSubcoreMesh(
        core_axis_name="c", subcore_axis_name="s",
        num_cores=_NC, num_subcores=_NS)


_SC_PARAMS = pltpu.CompilerParams(
    needs_layout_passes=False, use_tc_tiling_on_sc=False)


def _sc_scores(q, k, src, dst):
    @functools.partial(
        pl.kernel,
        out_type=(jax.ShapeDtypeStruct((_E,), jnp.float32),
                  jax.ShapeDtypeStruct((_NC, _DR, _H), jnp.float32)),
        mesh=_sc_mesh(),
        compiler_params=_SC_PARAMS,
        scratch_types=[
            pltpu.VMEM((_EPT,), jnp.int32),
            pltpu.VMEM((_EPT,), jnp.int32),
            pltpu.VMEM((_EB, _H), jnp.float32),
            pltpu.VMEM((_EB, _H), jnp.float32),
            pltpu.VMEM((_EB, _H), jnp.float32),
            pltpu.VMEM((_EB, _H), jnp.float32),
            pltpu.VMEM((_L, _L + 1), jnp.float32),
            pltpu.VMEM((_EPT,), jnp.float32),
            pltpu.VMEM((_DR, _H), jnp.float32),
            pltpu.VMEM((_DR,), jnp.int32),
            pltpu.VMEM_SHARED((_DR, _H), jnp.float32),
            pltpu.SemaphoreType.DMA,
            pltpu.SemaphoreType.DMA,
        ],
    )
    def run(q_h, k_h, src_h, dst_h, ex_h, den_h,
            src_v, dst_v, qra, kra, qrb, krb, pbuf, exbuf, dacc, irow,
            dspm, sma, smb):
        c = lax.axis_index("c")
        s = lax.axis_index("s")
        w = c * _NS + s
        base = w * _EPT
        pltpu.sync_copy(src_h.at[pl.ds(base, _EPT)], src_v)
        pltpu.sync_copy(dst_h.at[pl.ds(base, _EPT)], dst_v)

        def zrow(i, car):
            for d in range(_DL):
                dacc[i, pl.ds(d * _L, _L)] = jnp.zeros((_L,), jnp.float32)
            return car
        lax.fori_loop(0, _DR, zrow, 0)
        for t in range(_DR // _L):
            irow[pl.ds(t * _L, _L)] = lax.iota(jnp.int32, _L) + t * _L
        pltpu.sync_copy(dacc.at[pl.ds(s * _DRT, _DRT)],
                        dspm.at[pl.ds(s * _DRT, _DRT)])

        lane = lax.iota(jnp.int32, _L)

        def start(b, qr, kr, sem):
            off = b * _EB
            pltpu.async_copy(q_h.at[dst_v.at[pl.ds(off, _EB)]], qr, sem)
            pltpu.async_copy(k_h.at[src_v.at[pl.ds(off, _EB)]], kr, sem)

        def wait(qr, kr, sem):
            pltpu.make_async_copy(q_h.at[dst_v.at[pl.ds(0, _EB)]],
                                  qr, sem).wait()
            pltpu.make_async_copy(k_h.at[src_v.at[pl.ds(0, _EB)]],
                                  kr, sem).wait()

        def compute(b, qr, kr):
            boff = b * _EB

            def grp(g, car):
                goff = g * _L

                def edacc(j, car2):
                    e = goff + j
                    acc = qr[e, pl.ds(0, _L)] * kr[e, pl.ds(0, _L)]
                    for d in range(1, _DL):
                        acc = acc + (qr[e, pl.ds(d * _L, _L)] *
                                     kr[e, pl.ds(d * _L, _L)])
                    pbuf[j, pl.ds(0, _L)] = acc
                    return car2
                lax.fori_loop(0, _L, edacc, 0)

                ssum = jnp.zeros((_L,), jnp.float32)
                for i in range(_L):
                    ssum = ssum + plsc.load_gather(
                        pbuf, [lane, jnp.full((_L,), i, jnp.int32)])
                ex16 = jnp.exp(ssum * _ISQ)
                exbuf[pl.ds(boff + goff, _L)] = ex16
                d16 = dst_v[pl.ds(boff + goff, _L)]
                row = lax.shift_right_logical(d16, 7)
                col = lax.bitwise_and(d16, _H - 1)
                for r in range(_L):
                    plsc.addupdate_scatter(dacc, [row, col], ex16,
                                           mask=lane == r)
                return car
            lax.fori_loop(0, _EB // _L, grp, 0)

        start(0, qra, kra, sma)

        def pair(t, car):
            b = 2 * t
            start(b + 1, qrb, krb, smb)
            wait(qra, kra, sma)
            compute(b, qra, kra)
            start(b + 2, qra, kra, sma)
            wait(qrb, krb, smb)
            compute(b + 1, qrb, krb)
            return car
        lax.fori_loop(0, _NPAIR, pair, 0)
        wait(qra, kra, sma)
        compute(_NB - 1, qra, kra)

        pltpu.sync_copy(exbuf, ex_h.at[pl.ds(base, _EPT)])
        plsc.subcore_barrier()
        pltpu.sync_copy(dacc, dspm.at[irow], add=True)
        plsc.subcore_barrier()
        pltpu.sync_copy(dspm.at[pl.ds(s * _DRT, _DRT)],
                        den_h.at[c, pl.ds(s * _DRT, _DRT)])

    return run(q, k, src, dst)


def _sc_agg(v, src, dst, ex):
    @functools.partial(
        pl.kernel,
        out_type=jax.ShapeDtypeStruct((_NC, _NPAD, _H), jnp.float32),
        mesh=_sc_mesh(),
        compiler_params=_SC_PARAMS,
        scratch_types=[
            pltpu.VMEM((_EB,), jnp.int32),
            pltpu.VMEM((_EB,), jnp.int32),
            pltpu.VMEM((_EB,), jnp.float32),
            pltpu.VMEM((_EB,), jnp.int32),
            pltpu.VMEM((_EB,), jnp.int32),
            pltpu.VMEM((_EB,), jnp.float32),
            pltpu.VMEM((_EB, _H), jnp.float32),
            pltpu.VMEM((_EB, _H), jnp.float32),
            pltpu.VMEM((_EB, _H), jnp.float32),
            pltpu.VMEM((_L, _L + 1), jnp.float32),
            pltpu.VMEM_SHARED((_NPAD, _H), jnp.float32),
            pltpu.SemaphoreType.DMA,
            pltpu.SemaphoreType.DMA,
        ],
    )
    def run(v_h, src_h, dst_h, ex_h, agg_h,
            scura, dcura, exba, scurb, dcurb, exbb, vra, vrb, wr, arep,
            aspm, sma, smb):
        c = lax.axis_index("c")
        s = lax.axis_index("s")
        w = c * _NS + s
        base = w * _EPT
        lane = lax.iota(jnp.int32, _L)

        def zw(i, car):
            for d in range(_DL):
                wr[i, pl.ds(d * _L, _L)] = jnp.zeros((_L,), jnp.float32)
            return car
        lax.fori_loop(0, _EB, zw, 0)
        for j in range(_APT // _EB):
            pltpu.sync_copy(wr, aspm.at[pl.ds(s * _APT + j * _EB, _EB)])
        plsc.subcore_barrier()

        def loadidx(b, scur, dcur, exb):
            off = base + b * _EB
            pltpu.sync_copy(src_h.at[pl.ds(off, _EB)], scur)
            pltpu.sync_copy(dst_h.at[pl.ds(off, _EB)], dcur)
            pltpu.sync_copy(ex_h.at[pl.ds(off, _EB)], exb)

        def startv(scur, vr, sem):
            pltpu.async_copy(v_h.at[scur], vr, sem)

        def waitv(scur, vr, sem):
            pltpu.make_async_copy(v_h.at[scur], vr, sem).wait()

        def compute(dcur, exb, vr):
            def grp(g, car):
                goff = g * _L
                ex16 = exb[pl.ds(goff, _L)]
                for i in range(_L):
                    arep[i, pl.ds(0, _L)] = ex16

                def edge(j, car2):
                    e = goff + j
                    esp = plsc.load_gather(
                        arep, [lane, jnp.zeros((_L,), jnp.int32) + j])
                    for d in range(_DL):
                        wr[e, pl.ds(d * _L, _L)] = (
                            vr[e, pl.ds(d * _L, _L)] * esp)
                    return car2
                lax.fori_loop(0, _L, edge, 0)
                return car
            lax.fori_loop(0, _EB // _L, grp, 0)
            pltpu.sync_copy(wr, aspm.at[dcur], add=True)

        loadidx(0, scura, dcura, exba)
        startv(scura, vra, sma)
        loadidx(1, scurb, dcurb, exbb)

        def pair(t, car):
            b = 2 * t
            startv(scurb, vrb, smb)
            waitv(scura, vra, sma)
            compute(dcura, exba, vra)
            loadidx(b + 2, scura, dcura, exba)
            startv(scura, vra, sma)
            waitv(scurb, vrb, smb)
            compute(dcurb, exbb, vrb)
            loadidx(jnp.minimum(b + 3, _NB - 1), scurb, dcurb, exbb)
            return car
        lax.fori_loop(0, _NPAIR, pair, 0)
        waitv(scura, vra, sma)
        compute(dcura, exba, vra)

        plsc.subcore_barrier()
        pltpu.sync_copy(aspm.at[pl.ds(s * _APT, _APT)],
                        agg_h.at[c, pl.ds(s * _APT, _APT)])

    return run(v, src, dst, ex)


def _tc_qkvs(h, p):
    blk = 1000
    grid = _N // blk

    def body(h_ref, wq_ref, wk_ref, wv_ref, ws_ref,
             bq_ref, bk_ref, bv_ref, bs_ref,
             q_ref, k_ref, v_ref, s_ref):
        hb = h_ref[...]
        q_ref[...] = jnp.dot(hb, wq_ref[...],
                             preferred_element_type=jnp.float32) + bq_ref[...]
        k_ref[...] = jnp.dot(hb, wk_ref[...],
                             preferred_element_type=jnp.float32) + bk_ref[...]
        v_ref[...] = jnp.dot(hb, wv_ref[...],
                             preferred_element_type=jnp.float32) + bv_ref[...]
        s_ref[...] = jnp.dot(hb, ws_ref[...],
                             preferred_element_type=jnp.float32) + bs_ref[...]

    return pl.pallas_call(
        body,
        grid=(grid,),
        in_specs=[pl.BlockSpec((blk, _H), lambda i: (i, 0))]
        + [pl.BlockSpec((_H, _H), lambda i: (0, 0))] * 4
        + [pl.BlockSpec((1, _H), lambda i: (0, 0))] * 4,
        out_specs=[pl.BlockSpec((blk, _H), lambda i: (i, 0))] * 4,
        out_shape=[jax.ShapeDtypeStruct((_N, _H), jnp.float32)] * 4,
    )(h, p["Wq"], p["Wk"], p["Wv"], p["Ws"],
      p["bq"].reshape(1, _H), p["bk"].reshape(1, _H),
      p["bv"].reshape(1, _H), p["bs"].reshape(1, _H))


def _tc_combine(agg2, s_arr, dn, bn):
    def body(a_ref, s_ref, dn_ref, g_ref, b_ref, o_ref):
        t = (a_ref[0] + a_ref[1]) / dn_ref[...] + s_ref[...]
        m = jnp.mean(t, axis=0, keepdims=True)
        var = jnp.mean((t - m) ** 2, axis=0, keepdims=True)
        hn = g_ref[...] * (t - m) * lax.rsqrt(var + 1e-5) + b_ref[...]
        o_ref[...] = jnp.maximum(hn, 0.0)

    return pl.pallas_call(
        body,
        out_shape=jax.ShapeDtypeStruct((_N, _H), jnp.float32),
    )(agg2, s_arr, dn, bn["g"].reshape(1, _H), bn["b"].reshape(1, _H))


def _tc_pool(h, batch, lin):
    def body(h_ref, b_ref, w_ref, bl_ref, o_ref):
        seg = lax.broadcasted_iota(jnp.int32, (_G, _N), 0)
        mask = jnp.where(seg == b_ref[...], 1.0, 0.0)
        sums = jnp.dot(mask, h_ref[...], preferred_element_type=jnp.float32)
        counts = jnp.sum(mask, axis=1, keepdims=True)
        pooled = sums / jnp.maximum(counts, 1.0)
        o_ref[...] = jnp.dot(pooled, w_ref[...],
                             preferred_element_type=jnp.float32) + bl_ref[...]

    return pl.pallas_call(
        body,
        out_shape=jax.ShapeDtypeStruct((_G, _CLS), jnp.float32),
    )(h, batch.reshape(1, _N), lin["W"], lin["b"].reshape(1, _CLS))


def kernel(x, edge_index, batch, params):
    ei0, ei1 = edge_index[0], edge_index[1]
    layers = [(params["conv1"], params["bn1"], ei0)]
    for pc, pb in zip(params["conv_c"], params["bn_c"]):
        layers.append((pc, pb, ei1))
    for pc, pb in zip(params["convs"], params["bns"]):
        layers.append((pc, pb, ei0))
    h = x
    for p, bn, ei in layers:
        src, dst = ei[0], ei[1]
        q, k, v, s_arr = _tc_qkvs(h, p)
        ex, den = _sc_scores(q, k, src, dst)
        agg = _sc_agg(v, src, dst, ex)
        dn = (den[0] + den[1]).reshape(_NPAD)[:_N, None] + 1e-16
        h = _tc_combine(agg[:, :_N, :], s_arr, dn, bn)
    return _tc_pool(h, batch, params["lin"])


# speedup vs baseline: 8.6467x; 1.0296x over previous
import functools

import jax
import jax.numpy as jnp
from jax import lax
from jax.experimental import pallas as pl
from jax.experimental.pallas import tpu as pltpu
from jax.experimental.pallas import tpu_sc as plsc

_N = 10000
_E = 320000
_H = 128
_G = 64
_CLS = 10
_NC = 2
_NS = 16
_L = 16
_NW = _NC * _NS
_EPT = _E // _NW
_EB = 80
_NB = _EPT // _EB
_NPAIR = (_NB - 1) // 2
_DL = _H // _L
_DR = 80
_NPAD = _DR * _H
_DRT = _DR // _NS
_APT = _NPAD // _NS
_EB2 = 4000
_NB2 = _E // _EB2
_FPT = _H // _NW
_AGR = _N * _FPT // _L
_ISQ = 1.0 / float(_H) ** 0.5


def _sc_mesh():
    return plsc.VectorSubcoreMesh(
        core_axis_name="c", subcore_axis_name="s",
        num_cores=_NC, num_subcores=_NS)


_SC_PARAMS = pltpu.CompilerParams(
    needs_layout_passes=False, use_tc_tiling_on_sc=False)


def _sc_scores(q, k, src, dst):
    @functools.partial(
        pl.kernel,
        out_type=(jax.ShapeDtypeStruct((_E,), jnp.float32),
                  jax.ShapeDtypeStruct((_NC, _DR, _H), jnp.float32)),
        mesh=_sc_mesh(),
        compiler_params=_SC_PARAMS,
        scratch_types=[
            pltpu.VMEM((_EPT,), jnp.int32),
            pltpu.VMEM((_EPT,), jnp.int32),
            pltpu.VMEM((_EB, _H), jnp.float32),
            pltpu.VMEM((_EB, _H), jnp.float32),
            pltpu.VMEM((_EB, _H), jnp.float32),
            pltpu.VMEM((_EB, _H), jnp.float32),
            pltpu.VMEM((_L, _L + 1), jnp.float32),
            pltpu.VMEM((_EPT,), jnp.float32),
            pltpu.VMEM((_DR, _H), jnp.float32),
            pltpu.VMEM((_DR,), jnp.int32),
            pltpu.VMEM_SHARED((_DR, _H), jnp.float32),
            pltpu.SemaphoreType.DMA,
            pltpu.SemaphoreType.DMA,
        ],
    )
    def run(q_h, k_h, src_h, dst_h, ex_h, den_h,
            src_v, dst_v, qra, kra, qrb, krb, pbuf, exbuf, dacc, irow,
            dspm, sma, smb):
        c = lax.axis_index("c")
        s = lax.axis_index("s")
        w = c * _NS + s
        base = w * _EPT
        pltpu.sync_copy(src_h.at[pl.ds(base, _EPT)], src_v)
        pltpu.sync_copy(dst_h.at[pl.ds(base, _EPT)], dst_v)

        def zrow(i, car):
            for d in range(_DL):
                dacc[i, pl.ds(d * _L, _L)] = jnp.zeros((_L,), jnp.float32)
            return car
        lax.fori_loop(0, _DR, zrow, 0)
        for t in range(_DR // _L):
            irow[pl.ds(t * _L, _L)] = lax.iota(jnp.int32, _L) + t * _L
        pltpu.sync_copy(dacc.at[pl.ds(s * _DRT, _DRT)],
                        dspm.at[pl.ds(s * _DRT, _DRT)])

        lane = lax.iota(jnp.int32, _L)

        def start(b, qr, kr, sem):
            off = b * _EB
            pltpu.async_copy(q_h.at[dst_v.at[pl.ds(off, _EB)]], qr, sem)
            pltpu.async_copy(k_h.at[src_v.at[pl.ds(off, _EB)]], kr, sem)

        def wait(qr, kr, sem):
            pltpu.make_async_copy(q_h.at[dst_v.at[pl.ds(0, _EB)]],
                                  qr, sem).wait()
            pltpu.make_async_copy(k_h.at[src_v.at[pl.ds(0, _EB)]],
                                  kr, sem).wait()

        def compute(b, qr, kr):
            boff = b * _EB

            def grp(g, car):
                goff = g * _L

                def edacc(j, car2):
                    e = goff + j
                    acc = qr[e, pl.ds(0, _L)] * kr[e, pl.ds(0, _L)]
                    for d in range(1, _DL):
                        acc = acc + (qr[e, pl.ds(d * _L, _L)] *
                                     kr[e, pl.ds(d * _L, _L)])
                    pbuf[j, pl.ds(0, _L)] = acc
                    return car2
                lax.fori_loop(0, _L, edacc, 0)

                ssum = jnp.zeros((_L,), jnp.float32)
                for i in range(_L):
                    ssum = ssum + plsc.load_gather(
                        pbuf, [lane, jnp.full((_L,), i, jnp.int32)])
                ex16 = jnp.exp(ssum * _ISQ)
                exbuf[pl.ds(boff + goff, _L)] = ex16
                d16 = dst_v[pl.ds(boff + goff, _L)]
                row = lax.shift_right_logical(d16, 7)
                col = lax.bitwise_and(d16, _H - 1)
                plsc.addupdate_scatter(dacc, [row, col], ex16)
                return car
            lax.fori_loop(0, _EB // _L, grp, 0)

        start(0, qra, kra, sma)

        def pair(t, car):
            b = 2 * t
            start(b + 1, qrb, krb, smb)
            wait(qra, kra, sma)
            compute(b, qra, kra)
            start(b + 2, qra, kra, sma)
            wait(qrb, krb, smb)
            compute(b + 1, qrb, krb)
            return car
        lax.fori_loop(0, _NPAIR, pair, 0)
        wait(qra, kra, sma)
        compute(_NB - 1, qra, kra)

        pltpu.sync_copy(exbuf, ex_h.at[pl.ds(base, _EPT)])
        plsc.subcore_barrier()
        pltpu.sync_copy(dacc, dspm.at[irow], add=True)
        plsc.subcore_barrier()
        pltpu.sync_copy(dspm.at[pl.ds(s * _DRT, _DRT)],
                        den_h.at[c, pl.ds(s * _DRT, _DRT)])

    return run(q, k, src, dst)


def _sc_agg(vt, src, dst, ex):
    @functools.partial(
        pl.kernel,
        out_type=jax.ShapeDtypeStruct((_NW, _AGR, _L), jnp.float32),
        mesh=_sc_mesh(),
        compiler_params=_SC_PARAMS,
        scratch_types=[
            pltpu.VMEM((_EB2,), jnp.int32),
            pltpu.VMEM((_EB2,), jnp.int32),
            pltpu.VMEM((_EB2,), jnp.float32),
            pltpu.VMEM((_EB2,), jnp.int32),
            pltpu.VMEM((_EB2,), jnp.int32),
            pltpu.VMEM((_EB2,), jnp.float32),
            pltpu.VMEM((_FPT, _N), jnp.float32),
            pltpu.VMEM((_AGR, _L), jnp.float32),
            pltpu.SemaphoreType.DMA,
            pltpu.SemaphoreType.DMA,
        ],
    )
    def run(vt_h, src_h, dst_h, ex_h, agg_h,
            sa, da, ea, sb, db, eb, vloc, aggloc, sma, smb):
        c = lax.axis_index("c")
        s = lax.axis_index("s")
        w = c * _NS + s
        pltpu.sync_copy(vt_h.at[pl.ds(w * _FPT, _FPT)], vloc)

        def zrow(i, car):
            aggloc[i, :] = jnp.zeros((_L,), jnp.float32)
            return car
        lax.fori_loop(0, _AGR, zrow, 0)

        def start(b, sbuf, dbuf, ebuf, sem):
            off = b * _EB2
            pltpu.async_copy(src_h.at[pl.ds(off, _EB2)], sbuf, sem)
            pltpu.async_copy(dst_h.at[pl.ds(off, _EB2)], dbuf, sem)
            pltpu.async_copy(ex_h.at[pl.ds(off, _EB2)], ebuf, sem)

        def wait(sbuf, dbuf, ebuf, sem):
            pltpu.make_async_copy(src_h.at[pl.ds(0, _EB2)], sbuf, sem).wait()
            pltpu.make_async_copy(dst_h.at[pl.ds(0, _EB2)], dbuf, sem).wait()
            pltpu.make_async_copy(ex_h.at[pl.ds(0, _EB2)], ebuf, sem).wait()

        def compute(sbuf, dbuf, ebuf):
            def grp(g, car):
                goff = g * _L
                s16 = sbuf[pl.ds(goff, _L)]
                d16 = dbuf[pl.ds(goff, _L)]
                e16 = ebuf[pl.ds(goff, _L)]
                for j in range(_FPT):
                    vv = plsc.load_gather(
                        vloc, [jnp.full((_L,), j, jnp.int32), s16])
                    flat = d16 + (j * _N)
                    row = lax.shift_right_logical(flat, 4)
                    col = lax.bitwise_and(flat, _L - 1)
                    plsc.addupdate_scatter(aggloc, [row, col], e16 * vv)
                return car
            lax.fori_loop(0, _EB2 // _L, grp, 0)

        start(0, sa, da, ea, sma)

        def pair(t, car):
            b = 2 * t
            start(b + 1, sb, db, eb, smb)
            wait(sa, da, ea, sma)
            compute(sa, da, ea)
            start(jnp.minimum(b + 2, _NB2 - 1), sa, da, ea, sma)
            wait(sb, db, eb, smb)
            compute(sb, db, eb)
            return car
        lax.fori_loop(0, _NB2 // 2, pair, 0)
        wait(sa, da, ea, sma)

        pltpu.sync_copy(aggloc, agg_h.at[w])

    return run(vt, src, dst, ex)


def _tc_qkvs(h, p):
    blk = 1000
    grid = _N // blk

    def body(h_ref, wq_ref, wk_ref, wv_ref, ws_ref,
             bq_ref, bk_ref, bv_ref, bs_ref,
             q_ref, k_ref, v_ref, s_ref):
        hb = h_ref[...]
        q_ref[...] = jnp.dot(hb, wq_ref[...],
                             preferred_element_type=jnp.float32) + bq_ref[...]
        k_ref[...] = jnp.dot(hb, wk_ref[...],
                             preferred_element_type=jnp.float32) + bk_ref[...]
        v_ref[...] = jnp.dot(hb, wv_ref[...],
                             preferred_element_type=jnp.float32) + bv_ref[...]
        s_ref[...] = jnp.dot(hb, ws_ref[...],
                             preferred_element_type=jnp.float32) + bs_ref[...]

    return pl.pallas_call(
        body,
        grid=(grid,),
        in_specs=[pl.BlockSpec((blk, _H), lambda i: (i, 0))]
        + [pl.BlockSpec((_H, _H), lambda i: (0, 0))] * 4
        + [pl.BlockSpec((1, _H), lambda i: (0, 0))] * 4,
        out_specs=[pl.BlockSpec((blk, _H), lambda i: (i, 0))] * 4,
        out_shape=[jax.ShapeDtypeStruct((_N, _H), jnp.float32)] * 4,
    )(h, p["Wq"], p["Wk"], p["Wv"], p["Ws"],
      p["bq"].reshape(1, _H), p["bk"].reshape(1, _H),
      p["bv"].reshape(1, _H), p["bs"].reshape(1, _H))


def _tc_combine(agg, s_arr, dn, bn):
    def body(a_ref, s_ref, dn_ref, g_ref, b_ref, o_ref):
        t = a_ref[...] / dn_ref[...] + s_ref[...]
        m = jnp.mean(t, axis=0, keepdims=True)
        var = jnp.mean((t - m) ** 2, axis=0, keepdims=True)
        hn = g_ref[...] * (t - m) * lax.rsqrt(var + 1e-5) + b_ref[...]
        o_ref[...] = jnp.maximum(hn, 0.0)

    return pl.pallas_call(
        body,
        out_shape=jax.ShapeDtypeStruct((_N, _H), jnp.float32),
    )(agg, s_arr, dn, bn["g"].reshape(1, _H), bn["b"].reshape(1, _H))


def _tc_pool(h, batch, lin):
    def body(h_ref, b_ref, w_ref, bl_ref, o_ref):
        seg = lax.broadcasted_iota(jnp.int32, (_G, _N), 0)
        mask = jnp.where(seg == b_ref[...], 1.0, 0.0)
        sums = jnp.dot(mask, h_ref[...], preferred_element_type=jnp.float32)
        counts = jnp.sum(mask, axis=1, keepdims=True)
        pooled = sums / jnp.maximum(counts, 1.0)
        o_ref[...] = jnp.dot(pooled, w_ref[...],
                             preferred_element_type=jnp.float32) + bl_ref[...]

    return pl.pallas_call(
        body,
        out_shape=jax.ShapeDtypeStruct((_G, _CLS), jnp.float32),
    )(h, batch.reshape(1, _N), lin["W"], lin["b"].reshape(1, _CLS))


def kernel(x, edge_index, batch, params):
    ei0, ei1 = edge_index[0], edge_index[1]
    layers = [(params["conv1"], params["bn1"], ei0)]
    for pc, pb in zip(params["conv_c"], params["bn_c"]):
        layers.append((pc, pb, ei1))
    for pc, pb in zip(params["convs"], params["bns"]):
        layers.append((pc, pb, ei0))
    h = x
    for p, bn, ei in layers:
        src, dst = ei[0], ei[1]
        q, k, v, s_arr = _tc_qkvs(h, p)
        ex, den = _sc_scores(q, k, src, dst)
        agg_t = _sc_agg(v.T, src, dst, ex)
        dn = (den[0] + den[1]).reshape(_NPAD)[:_N, None] + 1e-16
        agg = agg_t.reshape(_H, _N).T
        h = _tc_combine(agg, s_arr, dn, bn)
    return _tc_pool(h, batch, params["lin"])


# speedup vs baseline: 8.6480x; 1.0001x over previous
import functools

import jax
import jax.numpy as jnp
from jax import lax
from jax.experimental import pallas as pl
from jax.experimental.pallas import tpu as pltpu
from jax.experimental.pallas import tpu_sc as plsc

_N = 10000
_E = 320000
_H = 128
_G = 64
_CLS = 10
_NC = 2
_NS = 16
_L = 16
_NW = _NC * _NS
_EPT = _E // _NW
_EB = 80
_NB = _EPT // _EB
_NPAIR = (_NB - 1) // 2
_DL = _H // _L
_DR = 80
_NPAD = _DR * _H
_DRT = _DR // _NS
_APT = _NPAD // _NS
_EB2 = 4000
_NB2 = _E // _EB2
_FPT = _H // _NW
_AGR = _N * _FPT // _L
_HP = _H // 2
_ISQ = 1.0 / float(_H) ** 0.5


def _sc_mesh():
    return plsc.VectorSubcoreMesh(
        core_axis_name="c", subcore_axis_name="s",
        num_cores=_NC, num_subcores=_NS)


_SC_PARAMS = pltpu.CompilerParams(
    needs_layout_passes=False, use_tc_tiling_on_sc=False)


def _sc_scores(q, k, src, dst):
    @functools.partial(
        pl.kernel,
        out_type=(jax.ShapeDtypeStruct((_E,), jnp.float32),
                  jax.ShapeDtypeStruct((_NC, _DR, _H), jnp.float32)),
        mesh=_sc_mesh(),
        compiler_params=_SC_PARAMS,
        scratch_types=[
            pltpu.VMEM((_EPT,), jnp.int32),
            pltpu.VMEM((_EPT,), jnp.int32),
            pltpu.VMEM((_EB, _H), jnp.float32),
            pltpu.VMEM((_EB, _H), jnp.float32),
            pltpu.VMEM((_EB, _H), jnp.float32),
            pltpu.VMEM((_EB, _H), jnp.float32),
            pltpu.VMEM((_L, _L + 1), jnp.float32),
            pltpu.VMEM((_EPT,), jnp.float32),
            pltpu.VMEM((_DR, _H), jnp.float32),
            pltpu.VMEM((_DR,), jnp.int32),
            pltpu.VMEM_SHARED((_DR, _H), jnp.float32),
            pltpu.SemaphoreType.DMA,
            pltpu.SemaphoreType.DMA,
        ],
    )
    def run(q_h, k_h, src_h, dst_h, ex_h, den_h,
            src_v, dst_v, qra, kra, qrb, krb, pbuf, exbuf, dacc, irow,
            dspm, sma, smb):
        c = lax.axis_index("c")
        s = lax.axis_index("s")
        w = c * _NS + s
        base = w * _EPT
        pltpu.sync_copy(src_h.at[pl.ds(base, _EPT)], src_v)
        pltpu.sync_copy(dst_h.at[pl.ds(base, _EPT)], dst_v)

        def zrow(i, car):
            for d in range(_DL):
                dacc[i, pl.ds(d * _L, _L)] = jnp.zeros((_L,), jnp.float32)
            return car
        lax.fori_loop(0, _DR, zrow, 0)
        for t in range(_DR // _L):
            irow[pl.ds(t * _L, _L)] = lax.iota(jnp.int32, _L) + t * _L
        pltpu.sync_copy(dacc.at[pl.ds(s * _DRT, _DRT)],
                        dspm.at[pl.ds(s * _DRT, _DRT)])

        lane = lax.iota(jnp.int32, _L)

        def start(b, qr, kr, sem):
            off = b * _EB
            pltpu.async_copy(q_h.at[dst_v.at[pl.ds(off, _EB)]], qr, sem)
            pltpu.async_copy(k_h.at[src_v.at[pl.ds(off, _EB)]], kr, sem)

        def wait(qr, kr, sem):
            pltpu.make_async_copy(q_h.at[dst_v.at[pl.ds(0, _EB)]],
                                  qr, sem).wait()
            pltpu.make_async_copy(k_h.at[src_v.at[pl.ds(0, _EB)]],
                                  kr, sem).wait()

        def compute(b, qr, kr):
            boff = b * _EB

            def grp(g, car):
                goff = g * _L

                def edacc(j, car2):
                    e = goff + j
                    acc = qr[e, pl.ds(0, _L)] * kr[e, pl.ds(0, _L)]
                    for d in range(1, _DL):
                        acc = acc + (qr[e, pl.ds(d * _L, _L)] *
                                     kr[e, pl.ds(d * _L, _L)])
                    pbuf[j, pl.ds(0, _L)] = acc
                    return car2
                lax.fori_loop(0, _L, edacc, 0)

                ssum = jnp.zeros((_L,), jnp.float32)
                for i in range(_L):
                    ssum = ssum + plsc.load_gather(
                        pbuf, [lane, jnp.full((_L,), i, jnp.int32)])
                ex16 = jnp.exp(ssum * _ISQ)
                exbuf[pl.ds(boff + goff, _L)] = ex16
                d16 = dst_v[pl.ds(boff + goff, _L)]
                row = lax.shift_right_logical(d16, 7)
                col = lax.bitwise_and(d16, _H - 1)
                plsc.addupdate_scatter(dacc, [row, col], ex16)
                return car
            lax.fori_loop(0, _EB // _L, grp, 0)

        start(0, qra, kra, sma)

        def pair(t, car):
            b = 2 * t
            start(b + 1, qrb, krb, smb)
            wait(qra, kra, sma)
            compute(b, qra, kra)
            start(b + 2, qra, kra, sma)
            wait(qrb, krb, smb)
            compute(b + 1, qrb, krb)
            return car
        lax.fori_loop(0, _NPAIR, pair, 0)
        wait(qra, kra, sma)
        compute(_NB - 1, qra, kra)

        pltpu.sync_copy(exbuf, ex_h.at[pl.ds(base, _EPT)])
        plsc.subcore_barrier()
        pltpu.sync_copy(dacc, dspm.at[irow], add=True)
        plsc.subcore_barrier()
        pltpu.sync_copy(dspm.at[pl.ds(s * _DRT, _DRT)],
                        den_h.at[c, pl.ds(s * _DRT, _DRT)])

    return run(q, k, src, dst)


def _sc_agg(vt, src, dst, ex):
    @functools.partial(
        pl.kernel,
        out_type=jax.ShapeDtypeStruct((_NW, _AGR, _L), jnp.float32),
        mesh=_sc_mesh(),
        compiler_params=_SC_PARAMS,
        scratch_types=[
            pltpu.VMEM((_EB2,), jnp.int32),
            pltpu.VMEM((_EB2,), jnp.int32),
            pltpu.VMEM((_EB2,), jnp.float32),
            pltpu.VMEM((_EB2,), jnp.int32),
            pltpu.VMEM((_EB2,), jnp.int32),
            pltpu.VMEM((_EB2,), jnp.float32),
            pltpu.VMEM((_FPT, _N), jnp.float32),
            pltpu.VMEM((_AGR, _L), jnp.float32),
            pltpu.SemaphoreType.DMA,
            pltpu.SemaphoreType.DMA,
        ],
    )
    def run(vt_h, src_h, dst_h, ex_h, agg_h,
            sa, da, ea, sb, db, eb, vloc, aggloc, sma, smb):
        c = lax.axis_index("c")
        s = lax.axis_index("s")
        w = c * _NS + s
        pltpu.sync_copy(vt_h.at[pl.ds(w * _FPT, _FPT)], vloc)

        def zrow(i, car):
            aggloc[i, :] = jnp.zeros((_L,), jnp.float32)
            return car
        lax.fori_loop(0, _AGR, zrow, 0)

        def start(b, sbuf, dbuf, ebuf, sem):
            off = b * _EB2
            pltpu.async_copy(src_h.at[pl.ds(off, _EB2)], sbuf, sem)
            pltpu.async_copy(dst_h.at[pl.ds(off, _EB2)], dbuf, sem)
            pltpu.async_copy(ex_h.at[pl.ds(off, _EB2)], ebuf, sem)

        def wait(sbuf, dbuf, ebuf, sem):
            pltpu.make_async_copy(src_h.at[pl.ds(0, _EB2)], sbuf, sem).wait()
            pltpu.make_async_copy(dst_h.at[pl.ds(0, _EB2)], dbuf, sem).wait()
            pltpu.make_async_copy(ex_h.at[pl.ds(0, _EB2)], ebuf, sem).wait()

        def compute(sbuf, dbuf, ebuf):
            def grp(g, car):
                goff = g * _L
                s16 = sbuf[pl.ds(goff, _L)]
                d16 = dbuf[pl.ds(goff, _L)]
                e16 = ebuf[pl.ds(goff, _L)]
                row = lax.shift_right_logical(d16, 4)
                col = lax.bitwise_and(d16, _L - 1)
                for j in range(_FPT):
                    vv = plsc.load_gather(
                        vloc, [jnp.full((_L,), j, jnp.int32), s16])
                    plsc.addupdate_scatter(
                        aggloc, [row + (j * (_N // _L)), col], e16 * vv)
                return car
            lax.fori_loop(0, _EB2 // _L, grp, 0)

        start(0, sa, da, ea, sma)

        def pair(t, car):
            b = 2 * t
            start(b + 1, sb, db, eb, smb)
            wait(sa, da, ea, sma)
            compute(sa, da, ea)
            start(jnp.minimum(b + 2, _NB2 - 1), sa, da, ea, sma)
            wait(sb, db, eb, smb)
            compute(sb, db, eb)
            return car
        lax.fori_loop(0, _NB2 // 2, pair, 0)
        wait(sa, da, ea, sma)

        pltpu.sync_copy(aggloc, agg_h.at[w])

    return run(vt, src, dst, ex)


def _tc_qkvs(h, p):
    blk = 1000
    grid = _N // blk

    def body(h_ref, wq_ref, wk_ref, wv_ref, ws_ref,
             bq_ref, bk_ref, bv_ref, bs_ref,
             q_ref, k_ref, v_ref, s_ref):
        hb = h_ref[...]
        q_ref[...] = jnp.dot(hb, wq_ref[...],
                             preferred_element_type=jnp.float32) + bq_ref[...]
        k_ref[...] = jnp.dot(hb, wk_ref[...],
                             preferred_element_type=jnp.float32) + bk_ref[...]
        v_ref[...] = jnp.dot(hb, wv_ref[...],
                             preferred_element_type=jnp.float32) + bv_ref[...]
        s_ref[...] = jnp.dot(hb, ws_ref[...],
                             preferred_element_type=jnp.float32) + bs_ref[...]

    return pl.pallas_call(
        body,
        grid=(grid,),
        in_specs=[pl.BlockSpec((blk, _H), lambda i: (i, 0))]
        + [pl.BlockSpec((_H, _H), lambda i: (0, 0))] * 4
        + [pl.BlockSpec((1, _H), lambda i: (0, 0))] * 4,
        out_specs=[pl.BlockSpec((blk, _H), lambda i: (i, 0))] * 4,
        out_shape=[jax.ShapeDtypeStruct((_N, _H), jnp.float32)] * 4,
    )(h, p["Wq"], p["Wk"], p["Wv"], p["Ws"],
      p["bq"].reshape(1, _H), p["bk"].reshape(1, _H),
      p["bv"].reshape(1, _H), p["bs"].reshape(1, _H))


def _tc_combine(agg, s_arr, dn, bn):
    def body(a_ref, s_ref, dn_ref, g_ref, b_ref, o_ref):
        t = a_ref[...] / dn_ref[...] + s_ref[...]
        m = jnp.mean(t, axis=0, keepdims=True)
        var = jnp.mean((t - m) ** 2, axis=0, keepdims=True)
        hn = g_ref[...] * (t - m) * lax.rsqrt(var + 1e-5) + b_ref[...]
        o_ref[...] = jnp.maximum(hn, 0.0)

    return pl.pallas_call(
        body,
        out_shape=jax.ShapeDtypeStruct((_N, _H), jnp.float32),
    )(agg, s_arr, dn, bn["g"].reshape(1, _H), bn["b"].reshape(1, _H))


def _tc_pool(h, batch, lin):
    def body(h_ref, b_ref, w_ref, bl_ref, o_ref):
        seg = lax.broadcasted_iota(jnp.int32, (_G, _N), 0)
        mask = jnp.where(seg == b_ref[...], 1.0, 0.0)
        sums = jnp.dot(mask, h_ref[...], preferred_element_type=jnp.float32)
        counts = jnp.sum(mask, axis=1, keepdims=True)
        pooled = sums / jnp.maximum(counts, 1.0)
        o_ref[...] = jnp.dot(pooled, w_ref[...],
                             preferred_element_type=jnp.float32) + bl_ref[...]

    return pl.pallas_call(
        body,
        out_shape=jax.ShapeDtypeStruct((_G, _CLS), jnp.float32),
    )(h, batch.reshape(1, _N), lin["W"], lin["b"].reshape(1, _CLS))


def kernel(x, edge_index, batch, params):
    ei0, ei1 = edge_index[0], edge_index[1]
    layers = [(params["conv1"], params["bn1"], ei0)]
    for pc, pb in zip(params["conv_c"], params["bn_c"]):
        layers.append((pc, pb, ei1))
    for pc, pb in zip(params["convs"], params["bns"]):
        layers.append((pc, pb, ei0))
    h = x
    for p, bn, ei in layers:
        src, dst = ei[0], ei[1]
        q, k, v, s_arr = _tc_qkvs(h, p)
        ex, den = _sc_scores(q, k, src, dst)
        agg_t = _sc_agg(v.T, src, dst, ex)
        dn = (den[0] + den[1]).reshape(_NPAD)[:_N, None] + 1e-16
        agg = agg_t.reshape(_H, _N).T
        h = _tc_combine(agg, s_arr, dn, bn)
    return _tc_pool(h, batch, params["lin"])


# speedup vs baseline: 8.6532x; 1.0006x over previous
import functools

import jax
import jax.numpy as jnp
from jax import lax
from jax.experimental import pallas as pl
from jax.experimental.pallas import tpu as pltpu
from jax.experimental.pallas import tpu_sc as plsc

_N = 10000
_E = 320000
_H = 128
_G = 64
_CLS = 10
_NC = 2
_NS = 16
_L = 16
_NW = _NC * _NS
_EPT = _E // _NW
_EB = 80
_NB = _EPT // _EB
_NPAIR = (_NB - 1) // 2
_DL = _H // _L
_DR = 80
_NPAD = _DR * _H
_DRT = _DR // _NS
_APT = _NPAD // _NS
_EB2 = 4000
_NB2 = _E // _EB2
_FPT = _H // _NW
_AGR = _N * _FPT // _L
_HP = _H // 2
_ISQ = 1.0 / float(_H) ** 0.5


def _sc_mesh():
    return plsc.VectorSubcoreMesh(
        core_axis_name="c", subcore_axis_name="s",
        num_cores=_NC, num_subcores=_NS)


_SC_PARAMS = pltpu.CompilerParams(
    needs_layout_passes=False, use_tc_tiling_on_sc=False)


def _sc_scores(q, k, src, dst):
    @functools.partial(
        pl.kernel,
        out_type=(jax.ShapeDtypeStruct((_E,), jnp.float32),
                  jax.ShapeDtypeStruct((_NC, _DR, _H), jnp.float32)),
        mesh=_sc_mesh(),
        compiler_params=_SC_PARAMS,
        scratch_types=[
            pltpu.VMEM((_EPT,), jnp.int32),
            pltpu.VMEM((_EPT,), jnp.int32),
            pltpu.VMEM((_EB, _H), jnp.float32),
            pltpu.VMEM((_EB, _H), jnp.float32),
            pltpu.VMEM((_EB, _H), jnp.float32),
            pltpu.VMEM((_EB, _H), jnp.float32),
            pltpu.VMEM((_L, _L + 1), jnp.float32),
            pltpu.VMEM((_EPT,), jnp.float32),
            pltpu.VMEM((_DR, _H), jnp.float32),
            pltpu.VMEM((_DR,), jnp.int32),
            pltpu.VMEM_SHARED((_DR, _H), jnp.float32),
            pltpu.SemaphoreType.DMA,
            pltpu.SemaphoreType.DMA,
        ],
    )
    def run(q_h, k_h, src_h, dst_h, ex_h, den_h,
            src_v, dst_v, qra, kra, qrb, krb, pbuf, exbuf, dacc, irow,
            dspm, sma, smb):
        c = lax.axis_index("c")
        s = lax.axis_index("s")
        w = c * _NS + s
        base = w * _EPT
        pltpu.sync_copy(src_h.at[pl.ds(base, _EPT)], src_v)
        pltpu.sync_copy(dst_h.at[pl.ds(base, _EPT)], dst_v)

        def zrow(i, car):
            for d in range(_DL):
                dacc[i, pl.ds(d * _L, _L)] = jnp.zeros((_L,), jnp.float32)
            return car
        lax.fori_loop(0, _DR, zrow, 0)
        for t in range(_DR // _L):
            irow[pl.ds(t * _L, _L)] = lax.iota(jnp.int32, _L) + t * _L
        pltpu.sync_copy(dacc.at[pl.ds(s * _DRT, _DRT)],
                        dspm.at[pl.ds(s * _DRT, _DRT)])

        lane = lax.iota(jnp.int32, _L)

        def start(b, qr, kr, sem):
            off = b * _EB
            pltpu.async_copy(q_h.at[dst_v.at[pl.ds(off, _EB)]], qr, sem)
            pltpu.async_copy(k_h.at[src_v.at[pl.ds(off, _EB)]], kr, sem)

        def wait(qr, kr, sem):
            pltpu.make_async_copy(q_h.at[dst_v.at[pl.ds(0, _EB)]],
                                  qr, sem).wait()
            pltpu.make_async_copy(k_h.at[src_v.at[pl.ds(0, _EB)]],
                                  kr, sem).wait()

        def compute(b, qr, kr):
            boff = b * _EB

            def grp(g, car):
                goff = g * _L

                def edacc(j, car2):
                    e = goff + j
                    acc = qr[e, pl.ds(0, _L)] * kr[e, pl.ds(0, _L)]
                    for d in range(1, _DL):
                        acc = acc + (qr[e, pl.ds(d * _L, _L)] *
                                     kr[e, pl.ds(d * _L, _L)])
                    pbuf[j, pl.ds(0, _L)] = acc
                    return car2
                lax.fori_loop(0, _L, edacc, 0, unroll=4)

                ssum = jnp.zeros((_L,), jnp.float32)
                for i in range(_L):
                    ssum = ssum + plsc.load_gather(
                        pbuf, [lane, jnp.full((_L,), i, jnp.int32)])
                ex16 = jnp.exp(ssum * _ISQ)
                exbuf[pl.ds(boff + goff, _L)] = ex16
                d16 = dst_v[pl.ds(boff + goff, _L)]
                row = lax.shift_right_logical(d16, 7)
                col = lax.bitwise_and(d16, _H - 1)
                plsc.addupdate_scatter(dacc, [row, col], ex16)
                return car
            lax.fori_loop(0, _EB // _L, grp, 0)

        start(0, qra, kra, sma)

        def pair(t, car):
            b = 2 * t
            start(b + 1, qrb, krb, smb)
            wait(qra, kra, sma)
            compute(b, qra, kra)
            start(b + 2, qra, kra, sma)
            wait(qrb, krb, smb)
            compute(b + 1, qrb, krb)
            return car
        lax.fori_loop(0, _NPAIR, pair, 0)
        wait(qra, kra, sma)
        compute(_NB - 1, qra, kra)

        pltpu.sync_copy(exbuf, ex_h.at[pl.ds(base, _EPT)])
        plsc.subcore_barrier()
        pltpu.sync_copy(dacc, dspm.at[irow], add=True)
        plsc.subcore_barrier()
        pltpu.sync_copy(dspm.at[pl.ds(s * _DRT, _DRT)],
                        den_h.at[c, pl.ds(s * _DRT, _DRT)])

    return run(q, k, src, dst)


def _sc_agg(vt, src, dst, ex):
    @functools.partial(
        pl.kernel,
        out_type=jax.ShapeDtypeStruct((_NW, _AGR, _L), jnp.float32),
        mesh=_sc_mesh(),
        compiler_params=_SC_PARAMS,
        scratch_types=[
            pltpu.VMEM((_EB2,), jnp.int32),
            pltpu.VMEM((_EB2,), jnp.int32),
            pltpu.VMEM((_EB2,), jnp.float32),
            pltpu.VMEM((_EB2,), jnp.int32),
            pltpu.VMEM((_EB2,), jnp.int32),
            pltpu.VMEM((_EB2,), jnp.float32),
            pltpu.VMEM((_FPT, _N), jnp.float32),
            pltpu.VMEM((_AGR, _L), jnp.float32),
            pltpu.SemaphoreType.DMA,
            pltpu.SemaphoreType.DMA,
        ],
    )
    def run(vt_h, src_h, dst_h, ex_h, agg_h,
            sa, da, ea, sb, db, eb, vloc, aggloc, sma, smb):
        c = lax.axis_index("c")
        s = lax.axis_index("s")
        w = c * _NS + s
        pltpu.sync_copy(vt_h.at[pl.ds(w * _FPT, _FPT)], vloc)

        def zrow(i, car):
            aggloc[i, :] = jnp.zeros((_L,), jnp.float32)
            return car
        lax.fori_loop(0, _AGR, zrow, 0)

        def start(b, sbuf, dbuf, ebuf, sem):
            off = b * _EB2
            pltpu.async_copy(src_h.at[pl.ds(off, _EB2)], sbuf, sem)
            pltpu.async_copy(dst_h.at[pl.ds(off, _EB2)], dbuf, sem)
            pltpu.async_copy(ex_h.at[pl.ds(off, _EB2)], ebuf, sem)

        def wait(sbuf, dbuf, ebuf, sem):
            pltpu.make_async_copy(src_h.at[pl.ds(0, _EB2)], sbuf, sem).wait()
            pltpu.make_async_copy(dst_h.at[pl.ds(0, _EB2)], dbuf, sem).wait()
            pltpu.make_async_copy(ex_h.at[pl.ds(0, _EB2)], ebuf, sem).wait()

        def compute(sbuf, dbuf, ebuf):
            def grp(g, car):
                goff = g * _L
                s16 = sbuf[pl.ds(goff, _L)]
                d16 = dbuf[pl.ds(goff, _L)]
                e16 = ebuf[pl.ds(goff, _L)]
                row = lax.shift_right_logical(d16, 4)
                col = lax.bitwise_and(d16, _L - 1)
                for j in range(_FPT):
                    vv = plsc.load_gather(
                        vloc, [jnp.full((_L,), j, jnp.int32), s16])
                    plsc.addupdate_scatter(
                        aggloc, [row + (j * (_N // _L)), col], e16 * vv)
                return car
            lax.fori_loop(0, _EB2 // _L, grp, 0, unroll=5)

        start(0, sa, da, ea, sma)

        def pair(t, car):
            b = 2 * t
            start(b + 1, sb, db, eb, smb)
            wait(sa, da, ea, sma)
            compute(sa, da, ea)
            start(jnp.minimum(b + 2, _NB2 - 1), sa, da, ea, sma)
            wait(sb, db, eb, smb)
            compute(sb, db, eb)
            return car
        lax.fori_loop(0, _NB2 // 2, pair, 0)
        wait(sa, da, ea, sma)

        pltpu.sync_copy(aggloc, agg_h.at[w])

    return run(vt, src, dst, ex)


def _tc_qkvs(h, p):
    blk = 1000
    grid = _N // blk

    def body(h_ref, wq_ref, wk_ref, wv_ref, ws_ref,
             bq_ref, bk_ref, bv_ref, bs_ref,
             q_ref, k_ref, v_ref, s_ref):
        hb = h_ref[...]
        q_ref[...] = jnp.dot(hb, wq_ref[...],
                             preferred_element_type=jnp.float32) + bq_ref[...]
        k_ref[...] = jnp.dot(hb, wk_ref[...],
                             preferred_element_type=jnp.float32) + bk_ref[...]
        v_ref[...] = jnp.dot(hb, wv_ref[...],
                             preferred_element_type=jnp.float32) + bv_ref[...]
        s_ref[...] = jnp.dot(hb, ws_ref[...],
                             preferred_element_type=jnp.float32) + bs_ref[...]

    return pl.pallas_call(
        body,
        grid=(grid,),
        in_specs=[pl.BlockSpec((blk, _H), lambda i: (i, 0))]
        + [pl.BlockSpec((_H, _H), lambda i: (0, 0))] * 4
        + [pl.BlockSpec((1, _H), lambda i: (0, 0))] * 4,
        out_specs=[pl.BlockSpec((blk, _H), lambda i: (i, 0))] * 4,
        out_shape=[jax.ShapeDtypeStruct((_N, _H), jnp.float32)] * 4,
    )(h, p["Wq"], p["Wk"], p["Wv"], p["Ws"],
      p["bq"].reshape(1, _H), p["bk"].reshape(1, _H),
      p["bv"].reshape(1, _H), p["bs"].reshape(1, _H))


def _tc_combine(agg, s_arr, dn, bn):
    def body(a_ref, s_ref, dn_ref, g_ref, b_ref, o_ref):
        t = a_ref[...] / dn_ref[...] + s_ref[...]
        m = jnp.mean(t, axis=0, keepdims=True)
        var = jnp.mean((t - m) ** 2, axis=0, keepdims=True)
        hn = g_ref[...] * (t - m) * lax.rsqrt(var + 1e-5) + b_ref[...]
        o_ref[...] = jnp.maximum(hn, 0.0)

    return pl.pallas_call(
        body,
        out_shape=jax.ShapeDtypeStruct((_N, _H), jnp.float32),
    )(agg, s_arr, dn, bn["g"].reshape(1, _H), bn["b"].reshape(1, _H))


def _tc_pool(h, batch, lin):
    def body(h_ref, b_ref, w_ref, bl_ref, o_ref):
        seg = lax.broadcasted_iota(jnp.int32, (_G, _N), 0)
        mask = jnp.where(seg == b_ref[...], 1.0, 0.0)
        sums = jnp.dot(mask, h_ref[...], preferred_element_type=jnp.float32)
        counts = jnp.sum(mask, axis=1, keepdims=True)
        pooled = sums / jnp.maximum(counts, 1.0)
        o_ref[...] = jnp.dot(pooled, w_ref[...],
                             preferred_element_type=jnp.float32) + bl_ref[...]

    return pl.pallas_call(
        body,
        out_shape=jax.ShapeDtypeStruct((_G, _CLS), jnp.float32),
    )(h, batch.reshape(1, _N), lin["W"], lin["b"].reshape(1, _CLS))


def kernel(x, edge_index, batch, params):
    ei0, ei1 = edge_index[0], edge_index[1]
    layers = [(params["conv1"], params["bn1"], ei0)]
    for pc, pb in zip(params["conv_c"], params["bn_c"]):
        layers.append((pc, pb, ei1))
    for pc, pb in zip(params["convs"], params["bns"]):
        layers.append((pc, pb, ei0))
    h = x
    for p, bn, ei in layers:
        src, dst = ei[0], ei[1]
        q, k, v, s_arr = _tc_qkvs(h, p)
        ex, den = _sc_scores(q, k, src, dst)
        agg_t = _sc_agg(v.T, src, dst, ex)
        dn = (den[0] + den[1]).reshape(_NPAD)[:_N, None] + 1e-16
        agg = agg_t.reshape(_H, _N).T
        h = _tc_combine(agg, s_arr, dn, bn)
    return _tc_pool(h, batch, params["lin"])
